# Initial kernel scaffold; baseline (speedup 1.0000x reference)
import jax
import jax.numpy as jnp
from jax.experimental import pallas as pl


def kernel(x, edge_index, edge_type, W0, b0, W1, b1, W2, b2):
    raise NotImplementedError("write your pallas kernel here")


# speedup vs baseline: 6.5835x; 6.5835x over previous
import functools

import jax
import jax.numpy as jnp
from jax import lax
from jax.experimental import pallas as pl
from jax.experimental.pallas import tpu as pltpu
from jax.experimental.pallas import tpu_sc as plsc

N_NODES = 10000
N_EDGES = 320000
D = 128
HALF = 5000
NC = 2
NS = 16
ES = N_EDGES // NS
SUB = 4000
NSUB = ES // SUB
CAP = ES + 128
G = 128
AR = 5120
DUMMY = HALF
NR = 312
ARC = 5008


def _make_hop(rel: int):
    mesh = plsc.VectorSubcoreMesh(core_axis_name="c", subcore_axis_name="s")

    @functools.partial(
        pl.kernel,
        out_type=jax.ShapeDtypeStruct((N_NODES, D), jnp.float32),
        mesh=mesh,
        compiler_params=pltpu.CompilerParams(needs_layout_passes=False),
        scratch_types=[
            pltpu.VMEM((SUB,), jnp.int32),
            pltpu.VMEM((SUB,), jnp.int32),
            pltpu.VMEM((SUB,), jnp.int32),
            pltpu.VMEM((CAP,), jnp.int32),
            pltpu.VMEM((G,), jnp.int32),
            pltpu.VMEM((G,), jnp.int32),
            pltpu.VMEM((G,), jnp.int32),
            pltpu.VMEM((G,), jnp.int32),
            pltpu.VMEM((G, D), jnp.float32),
            pltpu.VMEM((G, D), jnp.float32),
            pltpu.VMEM((ARC,), jnp.float32),
            pltpu.VMEM((336,), jnp.float32),
            pltpu.VMEM((320,), jnp.float32),
            pltpu.VMEM_SHARED((AR, D), jnp.float32),
            pltpu.VMEM_SHARED((NS * ARC,), jnp.float32),
            pltpu.SemaphoreType.DMA,
            pltpu.SemaphoreType.DMA,
        ],
    )
    def hop(cur_hbm, rows_hbm, cols_hbm, types_hbm, out_hbm,
            meta_r, meta_c, meta_t, clist, rstage_a, rstage_b,
            cstage_a, cstage_b, gbuf, gbuf_b, cntv, cbuf, tbuf,
            acc_sh, cntall_sh, sem_a, sem_b):
        c = lax.axis_index("c")
        s = lax.axis_index("s")
        lo = c * HALF

        zf16 = jnp.zeros((16,), jnp.float32)
        zi16 = jnp.zeros((16,), jnp.int32)
        ones16 = jnp.ones((16,), jnp.float32)

        def _zg(i, _):
            for k in range(8):
                gbuf[i, pl.ds(k * 16, 16)] = zf16
            return 0
        lax.fori_loop(0, G, _zg, 0)

        def _zc(i, _):
            cntv[pl.ds(i * 16, 16)] = zf16
            return 0
        lax.fori_loop(0, ARC // 16, _zc, 0)

        zbase = s * 320
        for off, sz in ((0, 128), (128, 128), (256, 64)):
            pltpu.sync_copy(gbuf.at[pl.ds(0, sz)],
                            acc_sh.at[pl.ds(zbase + off, sz)])
        plsc.subcore_barrier()

        offv = jnp.int32(0)
        for sub in range(NSUB):
            ebase = s * ES + sub * SUB
            pltpu.sync_copy(rows_hbm.at[pl.ds(ebase, SUB)], meta_r)
            pltpu.sync_copy(cols_hbm.at[pl.ds(ebase, SUB)], meta_c)
            pltpu.sync_copy(types_hbm.at[pl.ds(ebase, SUB)], meta_t)

            def _filt(g, off):
                ii = lax.iota(jnp.int32, 16)
                packed = jnp.zeros((16,), jnp.int32)
                comp = []
                for j in range(5):
                    sl = pl.ds(g * 80 + j * 16, 16)
                    rv = meta_r[sl]
                    cv = meta_c[sl]
                    tv = meta_t[sl]
                    m = (tv == rel) & (rv >= lo) & (rv < lo + HALF)
                    rl = jnp.where(m, rv - lo, DUMMY)
                    pk = jnp.where(m, cv * 8192 + (rv - lo), DUMMY)
                    keys = jnp.where(m, ii, ii + 16)
                    _, sp = plsc.sort_key_val(keys, pk)
                    comp.append(sp)
                    plsc.addupdate_scatter(cntv, [rl], ones16, mask=m)
                    packed = packed + jnp.where(m, 1 << (5 * j), 0)
                for st in (1, 2, 4, 8):
                    perm = jnp.bitwise_xor(ii, st)
                    _, sh = plsc.sort_key_val(perm, packed)
                    packed = packed + sh
                tot = packed[0]
                for j, sp in enumerate(comp):
                    clist[pl.ds(off, 16)] = sp
                    off = off + ((tot >> (5 * j)) & 31)
                return off
            offv = lax.fori_loop(0, SUB // 80, _filt, offv)

        for k in range(8):
            clist[pl.ds(offv + k * 16, 16)] = zi16 + DUMMY
        nb = (offv + (G - 1)) // G

        pltpu.sync_copy(cntv, cntall_sh.at[pl.ds(s * ARC, ARC)])

        def _unpack(b, rstage, cstage):
            gb = b * G
            for k in range(8):
                pk = clist[pl.ds(gb + k * 16, 16)]
                rstage[pl.ds(k * 16, 16)] = jnp.bitwise_and(pk, 8191)
                cstage[pl.ds(k * 16, 16)] = lax.shift_right_logical(pk, 13)

        @pl.when(nb > 0)
        def _prologue():
            _unpack(0, rstage_a, cstage_a)
            pltpu.async_copy(cur_hbm.at[cstage_a], gbuf, sem_a)

        def _gs(b, carry):
            def _one(buf, sem, cstage, rstage, obuf, osem, ostage, orstage):
                @pl.when(b + 1 < nb)
                def _prefetch():
                    _unpack(b + 1, orstage, ostage)
                    pltpu.async_copy(cur_hbm.at[ostage], obuf, osem)
                pltpu.make_async_copy(cur_hbm.at[cstage], buf, sem).wait()
                pltpu.sync_copy(buf, acc_sh.at[rstage], add=True)

            @pl.when(b % 2 == 0)
            def _even():
                _one(gbuf, sem_a, cstage_a, rstage_a,
                     gbuf_b, sem_b, cstage_b, rstage_b)

            @pl.when(b % 2 == 1)
            def _odd():
                _one(gbuf_b, sem_b, cstage_b, rstage_b,
                     gbuf, sem_a, cstage_a, rstage_a)
            return carry
        lax.fori_loop(0, nb, _gs, 0)

        plsc.subcore_barrier()

        nbase = s * NR
        for k in range(20):
            cbuf[pl.ds(k * 16, 16)] = zf16
        for tt in range(NS):
            pltpu.sync_copy(cntall_sh.at[pl.ds(tt * ARC + nbase, 320)], tbuf)
            for k in range(20):
                sl = pl.ds(k * 16, 16)
                cbuf[sl] = cbuf[sl] + tbuf[sl]

        def _norm_block(boff, rows):
            pltpu.sync_copy(acc_sh.at[pl.ds(nbase + boff, rows)],
                            gbuf.at[pl.ds(0, rows)])

            def _nrm(r, _):
                cvec = cbuf[pl.ds(boff + r, 16)]
                ivec = 1.0 / jnp.maximum(cvec, 1.0)
                bc = jnp.full((16,), ivec[0], jnp.float32)
                for k in range(8):
                    sl = pl.ds(k * 16, 16)
                    gbuf[r, sl] = gbuf[r, sl] * bc
                return 0
            lax.fori_loop(0, rows, _nrm, 0)
            pltpu.sync_copy(gbuf.at[pl.ds(0, rows)],
                            out_hbm.at[pl.ds(lo + nbase + boff, rows)])

        for boff, rows in ((0, 64), (64, 64), (128, 64), (192, 64), (256, 56)):
            _norm_block(boff, rows)

        @pl.when(s == NS - 1)
        def _tail():
            _norm_block(NR, 8)

    return hop


def _combine(oa, ob, oc, W0, b0, W1, b1, W2, b2):
    def body(oa_ref, ob_ref, oc_ref, w0_ref, w1_ref, w2_ref, bias_ref, out_ref):
        acc = jnp.dot(oa_ref[...], w0_ref[...],
                      preferred_element_type=jnp.float32,
                      precision=lax.Precision.HIGHEST)
        acc = acc + jnp.dot(ob_ref[...], w1_ref[...],
                            preferred_element_type=jnp.float32,
                            precision=lax.Precision.HIGHEST)
        acc = acc + jnp.dot(oc_ref[...], w2_ref[...],
                            preferred_element_type=jnp.float32,
                            precision=lax.Precision.HIGHEST)
        out_ref[...] = (acc + bias_ref[...]) * (1.0 / 3.0)

    bias = (b0 + b1 + b2).reshape(1, D)
    blk = 400
    grid = N_NODES // blk
    return pl.pallas_call(
        body,
        grid=(grid,),
        in_specs=[pl.BlockSpec((blk, D), lambda i: (i, 0))] * 3
        + [pl.BlockSpec((D, D), lambda i: (0, 0))] * 3
        + [pl.BlockSpec((1, D), lambda i: (0, 0))],
        out_specs=pl.BlockSpec((blk, D), lambda i: (i, 0)),
        out_shape=jax.ShapeDtypeStruct((N_NODES, D), jnp.float32),
    )(oa, ob, oc, W0, W1, W2, bias)


def kernel(x, edge_index, edge_type, W0, b0, W1, b1, W2, b2):
    rows = edge_index[0].astype(jnp.int32)
    cols = edge_index[1].astype(jnp.int32)
    et = edge_type.astype(jnp.int32)
    x = x.astype(jnp.float32)

    hops = {r: _make_hop(r) for r in range(4)}
    h0 = hops[0](x, rows, cols, et)
    h1 = hops[1](x, rows, cols, et)
    h2 = hops[2](x, rows, cols, et)
    o0 = hops[1](h0, rows, cols, et)
    o1 = hops[2](h1, rows, cols, et)
    o2 = hops[3](h2, rows, cols, et)
    return _combine(o0, o1, o2, W0, b0, W1, b1, W2, b2)


# speedup vs baseline: 7.3445x; 1.1156x over previous
import functools

import jax
import jax.numpy as jnp
from jax import lax
from jax.experimental import pallas as pl
from jax.experimental.pallas import tpu as pltpu
from jax.experimental.pallas import tpu_sc as plsc

N_NODES = 10000
N_EDGES = 320000
D = 128
HALF = 5000
NC = 2
NS = 16
ES = N_EDGES // NS
SUB = 4000
NSUB = ES // SUB
CAP = ES + 128
G = 128
AR = 5120
DUMMY = HALF
NR = 312
ARC = 5008


def _make_hop(rel: int, mode: str = "plain"):
    mesh = plsc.VectorSubcoreMesh(core_axis_name="c", subcore_axis_name="s")

    main_out = jax.ShapeDtypeStruct((N_NODES, D), jnp.float32)
    if mode == "save":
        outs = (main_out,
                jax.ShapeDtypeStruct((NC * NS * CAP,), jnp.int32),
                jax.ShapeDtypeStruct((NC * NS * 16,), jnp.int32),
                jax.ShapeDtypeStruct((NC * NS * ARC,), jnp.float32))
    else:
        outs = main_out

    @functools.partial(
        pl.kernel,
        out_type=outs,
        mesh=mesh,
        compiler_params=pltpu.CompilerParams(needs_layout_passes=False),
        scratch_types=[
            pltpu.VMEM((2 * SUB,), jnp.int32),
            pltpu.VMEM((2 * SUB,), jnp.int32),
            pltpu.VMEM((2 * SUB,), jnp.int32),
            pltpu.VMEM((CAP,), jnp.int32),
            pltpu.VMEM((G,), jnp.int32),
            pltpu.VMEM((G,), jnp.int32),
            pltpu.VMEM((G,), jnp.int32),
            pltpu.VMEM((G,), jnp.int32),
            pltpu.VMEM((G, D), jnp.float32),
            pltpu.VMEM((G, D), jnp.float32),
            pltpu.VMEM((ARC,), jnp.float32),
            pltpu.VMEM((336,), jnp.float32),
            pltpu.VMEM((320,), jnp.float32),
            pltpu.VMEM_SHARED((AR, D), jnp.float32),
            pltpu.VMEM_SHARED((NS * ARC,), jnp.float32),
            pltpu.SemaphoreType.DMA,
            pltpu.SemaphoreType.DMA,
            pltpu.SemaphoreType.DMA,
            pltpu.SemaphoreType.DMA,
        ],
    )
    def hop(cur_hbm, in1, in2, in3, out_hbm, *maybe_more):
        if mode == "save":
            rows_hbm, cols_hbm, types_hbm = in1, in2, in3
            (lists_hbm, offs_hbm, cnts_hbm, meta_r, meta_c, meta_t, clist,
             rstage_a, rstage_b, cstage_a, cstage_b, gbuf, gbuf_b, cntv,
             cbuf, tbuf, acc_sh, cntall_sh, sem_a, sem_b, sem_m,
             sem_s) = maybe_more
        elif mode == "use":
            lists_hbm, offs_hbm, cnts_hbm = in1, in2, in3
            (meta_r, meta_c, meta_t, clist, rstage_a, rstage_b, cstage_a,
             cstage_b, gbuf, gbuf_b, cntv, cbuf, tbuf, acc_sh, cntall_sh,
             sem_a, sem_b, sem_m, sem_s) = maybe_more
        else:
            rows_hbm, cols_hbm, types_hbm = in1, in2, in3
            (meta_r, meta_c, meta_t, clist, rstage_a, rstage_b, cstage_a,
             cstage_b, gbuf, gbuf_b, cntv, cbuf, tbuf, acc_sh, cntall_sh,
             sem_a, sem_b, sem_m, sem_s) = maybe_more
        c = lax.axis_index("c")
        s = lax.axis_index("s")
        lo = c * HALF

        zf16 = jnp.zeros((16,), jnp.float32)
        zi16 = jnp.zeros((16,), jnp.int32)
        ones16 = jnp.ones((16,), jnp.float32)

        def _zg(i, _):
            for k in range(8):
                gbuf[i, pl.ds(k * 16, 16)] = zf16
            return 0
        lax.fori_loop(0, 64, _zg, 0)

        def _zc(i, _):
            for k in range(4):
                cntv[pl.ds(i * 64 + k * 16, 16)] = zf16
            return 0
        if mode != "use":
            lax.fori_loop(0, ARC // 64, _zc, 0)
            cntv[pl.ds((ARC // 64) * 64, 16)] = zf16

        zbase = s * 320
        for k in range(5):
            pltpu.sync_copy(gbuf.at[pl.ds(0, 64)],
                            acc_sh.at[pl.ds(zbase + k * 64, 64)])
        plsc.subcore_barrier()

        wid = c * NS + s
        def _meta_fetch(sub, half):
            ebase = s * ES + sub * SUB
            hb = half * SUB
            pltpu.async_copy(rows_hbm.at[pl.ds(ebase, SUB)],
                             meta_r.at[pl.ds(hb, SUB)], sem_m)
            pltpu.async_copy(cols_hbm.at[pl.ds(ebase, SUB)],
                             meta_c.at[pl.ds(hb, SUB)], sem_m)
            pltpu.async_copy(types_hbm.at[pl.ds(ebase, SUB)],
                             meta_t.at[pl.ds(hb, SUB)], sem_m)

        def _meta_wait(half):
            hb = half * SUB
            ebase = s * ES
            pltpu.make_async_copy(rows_hbm.at[pl.ds(ebase, SUB)],
                                  meta_r.at[pl.ds(hb, SUB)], sem_m).wait()
            pltpu.make_async_copy(cols_hbm.at[pl.ds(ebase, SUB)],
                                  meta_c.at[pl.ds(hb, SUB)], sem_m).wait()
            pltpu.make_async_copy(types_hbm.at[pl.ds(ebase, SUB)],
                                  meta_t.at[pl.ds(hb, SUB)], sem_m).wait()

        offv = jnp.int32(0)
        if mode != "use":
            _meta_fetch(0, 0)
        for sub in range(NSUB if mode != "use" else 0):
            half = sub % 2
            _meta_wait(half)
            if sub + 1 < NSUB:
                _meta_fetch(sub + 1, 1 - half)

            def _filt(g, off):
                ii = lax.iota(jnp.int32, 16)
                packed = jnp.zeros((16,), jnp.int32)
                comp = []
                for j in range(5):
                    sl = pl.ds(half * SUB + g * 80 + j * 16, 16)
                    rv = meta_r[sl]
                    cv = meta_c[sl]
                    tv = meta_t[sl]
                    m = (tv == rel) & (rv >= lo) & (rv < lo + HALF)
                    rl = jnp.where(m, rv - lo, DUMMY)
                    pk = jnp.where(m, cv * 8192 + (rv - lo), DUMMY)
                    keys = jnp.where(m, ii, ii + 16)
                    _, sp = plsc.sort_key_val(keys, pk)
                    comp.append(sp)
                    plsc.addupdate_scatter(cntv, [rl], ones16, mask=m)
                    packed = packed + jnp.where(m, 1 << (5 * j), 0)
                for st in (1, 2, 4, 8):
                    perm = jnp.bitwise_xor(ii, st)
                    _, sh = plsc.sort_key_val(perm, packed)
                    packed = packed + sh
                tot = packed[0]
                for j, sp in enumerate(comp):
                    clist[pl.ds(off, 16)] = sp
                    off = off + ((tot >> (5 * j)) & 31)
                return off
            offv = lax.fori_loop(0, SUB // 80, _filt, offv)

        if mode == "use":
            pltpu.sync_copy(lists_hbm.at[pl.ds(wid * CAP, CAP)], clist)
            pltpu.sync_copy(offs_hbm.at[pl.ds(wid * 16, 16)],
                            rstage_a.at[pl.ds(0, 16)])
            offv = rstage_a[pl.ds(0, 16)][0]
        else:
            for k in range(8):
                clist[pl.ds(offv + k * 16, 16)] = zi16 + DUMMY
        nb = (offv + (G - 1)) // G

        if mode != "use":
            pltpu.sync_copy(cntv, cntall_sh.at[pl.ds(s * ARC, ARC)])
        if mode == "save":
            pltpu.sync_copy(clist, lists_hbm.at[pl.ds(wid * CAP, CAP)])
            rstage_a[pl.ds(0, 16)] = jnp.full((16,), offv, jnp.int32)
            pltpu.sync_copy(rstage_a.at[pl.ds(0, 16)],
                            offs_hbm.at[pl.ds(wid * 16, 16)])
            pltpu.sync_copy(cntv, cnts_hbm.at[pl.ds(wid * ARC, ARC)])

        def _unpack(b, rstage, cstage):
            gb = b * G
            for k in range(8):
                pk = clist[pl.ds(gb + k * 16, 16)]
                rstage[pl.ds(k * 16, 16)] = jnp.bitwise_and(pk, 8191)
                cstage[pl.ds(k * 16, 16)] = lax.shift_right_logical(pk, 13)

        @pl.when(nb > 0)
        def _prologue():
            _unpack(0, rstage_a, cstage_a)
            pltpu.async_copy(cur_hbm.at[cstage_a], gbuf, sem_a)

        def _gs(b, carry):
            def _one(buf, sem, cstage, rstage, obuf, osem, ostage, orstage):
                @pl.when(b >= 1)
                def _drain_prev():
                    pltpu.make_async_copy(obuf, acc_sh.at[orstage],
                                          sem_s).wait()

                @pl.when(b + 1 < nb)
                def _prefetch():
                    _unpack(b + 1, orstage, ostage)
                    pltpu.async_copy(cur_hbm.at[ostage], obuf, osem)
                pltpu.make_async_copy(cur_hbm.at[cstage], buf, sem).wait()
                pltpu.async_copy(buf, acc_sh.at[rstage], sem_s, add=True)

            @pl.when(b % 2 == 0)
            def _even():
                _one(gbuf, sem_a, cstage_a, rstage_a,
                     gbuf_b, sem_b, cstage_b, rstage_b)

            @pl.when(b % 2 == 1)
            def _odd():
                _one(gbuf_b, sem_b, cstage_b, rstage_b,
                     gbuf, sem_a, cstage_a, rstage_a)
            return carry
        lax.fori_loop(0, nb, _gs, 0)

        @pl.when(nb > 0)
        def _drain_last():
            pltpu.make_async_copy(gbuf, acc_sh.at[rstage_a], sem_s).wait()

        plsc.subcore_barrier()

        nbase = s * NR
        for k in range(20):
            cbuf[pl.ds(k * 16, 16)] = zf16
        for tt in range(NS):
            if mode == "use":
                pltpu.sync_copy(
                    cnts_hbm.at[pl.ds((c * NS + tt) * ARC + nbase, 320)], tbuf)
            else:
                pltpu.sync_copy(cntall_sh.at[pl.ds(tt * ARC + nbase, 320)],
                                tbuf)
            for k in range(20):
                sl = pl.ds(k * 16, 16)
                cbuf[sl] = cbuf[sl] + tbuf[sl]

        def _norm_block(boff, rows):
            pltpu.sync_copy(acc_sh.at[pl.ds(nbase + boff, rows)],
                            gbuf.at[pl.ds(0, rows)])

            def _nrm(r, _):
                cvec = cbuf[pl.ds(boff + r, 16)]
                ivec = 1.0 / jnp.maximum(cvec, 1.0)
                bc = jnp.full((16,), ivec[0], jnp.float32)
                for k in range(8):
                    sl = pl.ds(k * 16, 16)
                    gbuf[r, sl] = gbuf[r, sl] * bc
                return 0
            lax.fori_loop(0, rows, _nrm, 0)
            pltpu.sync_copy(gbuf.at[pl.ds(0, rows)],
                            out_hbm.at[pl.ds(lo + nbase + boff, rows)])

        for boff, rows in ((0, 64), (64, 64), (128, 64), (192, 64), (256, 56)):
            _norm_block(boff, rows)

        @pl.when(s == NS - 1)
        def _tail():
            _norm_block(NR, 8)

    return hop


def _combine(oa, ob, oc, W0, b0, W1, b1, W2, b2):
    def body(oa_ref, ob_ref, oc_ref, w0_ref, w1_ref, w2_ref, bias_ref, out_ref):
        acc = jnp.dot(oa_ref[...], w0_ref[...],
                      preferred_element_type=jnp.float32,
                      precision=lax.Precision.HIGHEST)
        acc = acc + jnp.dot(ob_ref[...], w1_ref[...],
                            preferred_element_type=jnp.float32,
                            precision=lax.Precision.HIGHEST)
        acc = acc + jnp.dot(oc_ref[...], w2_ref[...],
                            preferred_element_type=jnp.float32,
                            precision=lax.Precision.HIGHEST)
        out_ref[...] = (acc + bias_ref[...]) * (1.0 / 3.0)

    bias = (b0 + b1 + b2).reshape(1, D)
    blk = 400
    grid = N_NODES // blk
    return pl.pallas_call(
        body,
        grid=(grid,),
        in_specs=[pl.BlockSpec((blk, D), lambda i: (i, 0))] * 3
        + [pl.BlockSpec((D, D), lambda i: (0, 0))] * 3
        + [pl.BlockSpec((1, D), lambda i: (0, 0))],
        out_specs=pl.BlockSpec((blk, D), lambda i: (i, 0)),
        out_shape=jax.ShapeDtypeStruct((N_NODES, D), jnp.float32),
    )(oa, ob, oc, W0, W1, W2, bias)


def kernel(x, edge_index, edge_type, W0, b0, W1, b1, W2, b2):
    rows = edge_index[0].astype(jnp.int32)
    cols = edge_index[1].astype(jnp.int32)
    et = edge_type.astype(jnp.int32)
    x = x.astype(jnp.float32)

    h0 = _make_hop(0)(x, rows, cols, et)
    h1, l1, f1, c1 = _make_hop(1, "save")(x, rows, cols, et)
    h2, l2, f2, c2 = _make_hop(2, "save")(x, rows, cols, et)
    o0 = _make_hop(1, "use")(h0, l1, f1, c1)
    o1 = _make_hop(2, "use")(h1, l2, f2, c2)
    o2 = _make_hop(3)(h2, rows, cols, et)
    return _combine(o0, o1, o2, W0, b0, W1, b1, W2, b2)


# speedup vs baseline: 7.3561x; 1.0016x over previous
import functools

import jax
import jax.numpy as jnp
from jax import lax
from jax.experimental import pallas as pl
from jax.experimental.pallas import tpu as pltpu
from jax.experimental.pallas import tpu_sc as plsc

N_NODES = 10000
N_EDGES = 320000
D = 128
HALF = 5000
NC = 2
NS = 16
ES = N_EDGES // NS
SUB = 4000
NSUB = ES // SUB
CAP = ES + 128
G = 128
AR = 5120
DUMMY = HALF
NR = 312
ARC = 5008


def _make_hop(rel: int, mode: str = "plain"):
    mesh = plsc.VectorSubcoreMesh(core_axis_name="c", subcore_axis_name="s")

    main_out = jax.ShapeDtypeStruct((N_NODES, D), jnp.float32)
    if mode == "save":
        outs = (main_out,
                jax.ShapeDtypeStruct((NC * NS * CAP,), jnp.int32),
                jax.ShapeDtypeStruct((NC * NS * 16,), jnp.int32),
                jax.ShapeDtypeStruct((NC * NS * ARC,), jnp.float32))
    else:
        outs = main_out

    @functools.partial(
        pl.kernel,
        out_type=outs,
        mesh=mesh,
        compiler_params=pltpu.CompilerParams(needs_layout_passes=False),
        scratch_types=[
            pltpu.VMEM((2 * SUB,), jnp.int32),
            pltpu.VMEM((2 * SUB,), jnp.int32),
            pltpu.VMEM((2 * SUB,), jnp.int32),
            pltpu.VMEM((CAP,), jnp.int32),
            pltpu.VMEM((G,), jnp.int32),
            pltpu.VMEM((G,), jnp.int32),
            pltpu.VMEM((G,), jnp.int32),
            pltpu.VMEM((G,), jnp.int32),
            pltpu.VMEM((G, D), jnp.float32),
            pltpu.VMEM((G, D), jnp.float32),
            pltpu.VMEM((ARC,), jnp.float32),
            pltpu.VMEM((336,), jnp.float32),
            pltpu.VMEM((320,), jnp.float32),
            pltpu.VMEM_SHARED((AR, D), jnp.float32),
            pltpu.VMEM_SHARED((NS * ARC,), jnp.float32),
            pltpu.SemaphoreType.DMA,
            pltpu.SemaphoreType.DMA,
            pltpu.SemaphoreType.DMA,
        ],
    )
    def hop(cur_hbm, in1, in2, in3, out_hbm, *maybe_more):
        if mode == "save":
            rows_hbm, cols_hbm, types_hbm = in1, in2, in3
            (lists_hbm, offs_hbm, cnts_hbm, meta_r, meta_c, meta_t, clist,
             rstage_a, rstage_b, cstage_a, cstage_b, gbuf, gbuf_b, cntv,
             cbuf, tbuf, acc_sh, cntall_sh, sem_a, sem_b, sem_m) = maybe_more
        elif mode == "use":
            lists_hbm, offs_hbm, cnts_hbm = in1, in2, in3
            (meta_r, meta_c, meta_t, clist, rstage_a, rstage_b, cstage_a,
             cstage_b, gbuf, gbuf_b, cntv, cbuf, tbuf, acc_sh, cntall_sh,
             sem_a, sem_b, sem_m) = maybe_more
        else:
            rows_hbm, cols_hbm, types_hbm = in1, in2, in3
            (meta_r, meta_c, meta_t, clist, rstage_a, rstage_b, cstage_a,
             cstage_b, gbuf, gbuf_b, cntv, cbuf, tbuf, acc_sh, cntall_sh,
             sem_a, sem_b, sem_m) = maybe_more
        c = lax.axis_index("c")
        s = lax.axis_index("s")
        lo = c * HALF

        zf16 = jnp.zeros((16,), jnp.float32)
        zi16 = jnp.zeros((16,), jnp.int32)
        ones16 = jnp.ones((16,), jnp.float32)

        def _zg(i, _):
            for k in range(8):
                gbuf[i, pl.ds(k * 16, 16)] = zf16
            return 0
        lax.fori_loop(0, 64, _zg, 0)

        def _zc(i, _):
            for k in range(4):
                cntv[pl.ds(i * 64 + k * 16, 16)] = zf16
            return 0
        if mode != "use":
            lax.fori_loop(0, ARC // 64, _zc, 0)
            cntv[pl.ds((ARC // 64) * 64, 16)] = zf16

        zbase = s * 320
        for k in range(5):
            pltpu.sync_copy(gbuf.at[pl.ds(0, 64)],
                            acc_sh.at[pl.ds(zbase + k * 64, 64)])
        plsc.subcore_barrier()

        wid = c * NS + s
        def _meta_fetch(sub, half):
            ebase = s * ES + sub * SUB
            hb = half * SUB
            pltpu.async_copy(rows_hbm.at[pl.ds(ebase, SUB)],
                             meta_r.at[pl.ds(hb, SUB)], sem_m)
            pltpu.async_copy(cols_hbm.at[pl.ds(ebase, SUB)],
                             meta_c.at[pl.ds(hb, SUB)], sem_m)
            pltpu.async_copy(types_hbm.at[pl.ds(ebase, SUB)],
                             meta_t.at[pl.ds(hb, SUB)], sem_m)

        def _meta_wait(half):
            hb = half * SUB
            ebase = s * ES
            pltpu.make_async_copy(rows_hbm.at[pl.ds(ebase, SUB)],
                                  meta_r.at[pl.ds(hb, SUB)], sem_m).wait()
            pltpu.make_async_copy(cols_hbm.at[pl.ds(ebase, SUB)],
                                  meta_c.at[pl.ds(hb, SUB)], sem_m).wait()
            pltpu.make_async_copy(types_hbm.at[pl.ds(ebase, SUB)],
                                  meta_t.at[pl.ds(hb, SUB)], sem_m).wait()

        offv = jnp.int32(0)
        if mode != "use":
            _meta_fetch(0, 0)
        for sub in range(NSUB if mode != "use" else 0):
            half = sub % 2
            _meta_wait(half)
            if sub + 1 < NSUB:
                _meta_fetch(sub + 1, 1 - half)

            def _filt(g, off):
                ii = lax.iota(jnp.int32, 16)
                packed = jnp.zeros((16,), jnp.int32)
                comp = []
                for j in range(5):
                    sl = pl.ds(half * SUB + g * 80 + j * 16, 16)
                    rv = meta_r[sl]
                    cv = meta_c[sl]
                    tv = meta_t[sl]
                    m = (tv == rel) & (rv >= lo) & (rv < lo + HALF)
                    rl = jnp.where(m, rv - lo, DUMMY)
                    pk = jnp.where(m, cv * 8192 + (rv - lo), DUMMY)
                    keys = jnp.where(m, ii, ii + 16)
                    _, sp = plsc.sort_key_val(keys, pk)
                    comp.append(sp)
                    plsc.addupdate_scatter(cntv, [rl], ones16, mask=m)
                    packed = packed + jnp.where(m, 1 << (5 * j), 0)
                for st in (1, 2, 4, 8):
                    perm = jnp.bitwise_xor(ii, st)
                    _, sh = plsc.sort_key_val(perm, packed)
                    packed = packed + sh
                tot = packed[0]
                for j, sp in enumerate(comp):
                    clist[pl.ds(off, 16)] = sp
                    off = off + ((tot >> (5 * j)) & 31)
                return off
            offv = lax.fori_loop(0, SUB // 80, _filt, offv)

        if mode == "use":
            pltpu.sync_copy(lists_hbm.at[pl.ds(wid * CAP, CAP)], clist)
            pltpu.sync_copy(offs_hbm.at[pl.ds(wid * 16, 16)],
                            rstage_a.at[pl.ds(0, 16)])
            offv = rstage_a[pl.ds(0, 16)][0]
        else:
            for k in range(8):
                clist[pl.ds(offv + k * 16, 16)] = zi16 + DUMMY
        nb = (offv + (G - 1)) // G

        if mode != "use":
            pltpu.sync_copy(cntv, cntall_sh.at[pl.ds(s * ARC, ARC)])
        if mode == "save":
            pltpu.sync_copy(clist, lists_hbm.at[pl.ds(wid * CAP, CAP)])
            rstage_a[pl.ds(0, 16)] = jnp.full((16,), offv, jnp.int32)
            pltpu.sync_copy(rstage_a.at[pl.ds(0, 16)],
                            offs_hbm.at[pl.ds(wid * 16, 16)])
            pltpu.sync_copy(cntv, cnts_hbm.at[pl.ds(wid * ARC, ARC)])

        def _unpack(b, rstage, cstage):
            gb = b * G
            for k in range(8):
                pk = clist[pl.ds(gb + k * 16, 16)]
                rstage[pl.ds(k * 16, 16)] = jnp.bitwise_and(pk, 8191)
                cstage[pl.ds(k * 16, 16)] = lax.shift_right_logical(pk, 13)

        @pl.when(nb > 0)
        def _prologue():
            _unpack(0, rstage_a, cstage_a)
            pltpu.async_copy(cur_hbm.at[cstage_a], gbuf, sem_a)

        def _gs(b, carry):
            def _one(buf, sem, cstage, rstage, obuf, osem, ostage, orstage):
                @pl.when(b + 1 < nb)
                def _prefetch():
                    _unpack(b + 1, orstage, ostage)
                    pltpu.async_copy(cur_hbm.at[ostage], obuf, osem)
                pltpu.make_async_copy(cur_hbm.at[cstage], buf, sem).wait()
                pltpu.sync_copy(buf, acc_sh.at[rstage], add=True)

            @pl.when(b % 2 == 0)
            def _even():
                _one(gbuf, sem_a, cstage_a, rstage_a,
                     gbuf_b, sem_b, cstage_b, rstage_b)

            @pl.when(b % 2 == 1)
            def _odd():
                _one(gbuf_b, sem_b, cstage_b, rstage_b,
                     gbuf, sem_a, cstage_a, rstage_a)
            return carry
        lax.fori_loop(0, nb, _gs, 0)

        plsc.subcore_barrier()

        nbase = s * NR
        for k in range(20):
            cbuf[pl.ds(k * 16, 16)] = zf16
        for tt in range(NS):
            if mode == "use":
                pltpu.sync_copy(
                    cnts_hbm.at[pl.ds((c * NS + tt) * ARC + nbase, 320)], tbuf)
            else:
                pltpu.sync_copy(cntall_sh.at[pl.ds(tt * ARC + nbase, 320)],
                                tbuf)
            for k in range(20):
                sl = pl.ds(k * 16, 16)
                cbuf[sl] = cbuf[sl] + tbuf[sl]

        def _norm_block(boff, rows):
            pltpu.sync_copy(acc_sh.at[pl.ds(nbase + boff, rows)],
                            gbuf.at[pl.ds(0, rows)])

            def _nrm(r, _):
                cvec = cbuf[pl.ds(boff + r, 16)]
                ivec = 1.0 / jnp.maximum(cvec, 1.0)
                bc = jnp.full((16,), ivec[0], jnp.float32)
                for k in range(8):
                    sl = pl.ds(k * 16, 16)
                    gbuf[r, sl] = gbuf[r, sl] * bc
                return 0
            lax.fori_loop(0, rows, _nrm, 0)
            pltpu.sync_copy(gbuf.at[pl.ds(0, rows)],
                            out_hbm.at[pl.ds(lo + nbase + boff, rows)])

        for boff, rows in ((0, 64), (64, 64), (128, 64), (192, 64), (256, 56)):
            _norm_block(boff, rows)

        @pl.when(s == NS - 1)
        def _tail():
            _norm_block(NR, 8)

    return hop


def _combine(oa, ob, oc, W0, b0, W1, b1, W2, b2):
    def body(oa_ref, ob_ref, oc_ref, w0_ref, w1_ref, w2_ref, bias_ref, out_ref):
        acc = jnp.dot(oa_ref[...], w0_ref[...],
                      preferred_element_type=jnp.float32,
                      precision=lax.Precision.HIGHEST)
        acc = acc + jnp.dot(ob_ref[...], w1_ref[...],
                            preferred_element_type=jnp.float32,
                            precision=lax.Precision.HIGHEST)
        acc = acc + jnp.dot(oc_ref[...], w2_ref[...],
                            preferred_element_type=jnp.float32,
                            precision=lax.Precision.HIGHEST)
        out_ref[...] = (acc + bias_ref[...]) * (1.0 / 3.0)

    bias = (b0 + b1 + b2).reshape(1, D)
    blk = 400
    grid = N_NODES // blk
    return pl.pallas_call(
        body,
        grid=(grid,),
        in_specs=[pl.BlockSpec((blk, D), lambda i: (i, 0))] * 3
        + [pl.BlockSpec((D, D), lambda i: (0, 0))] * 3
        + [pl.BlockSpec((1, D), lambda i: (0, 0))],
        out_specs=pl.BlockSpec((blk, D), lambda i: (i, 0)),
        out_shape=jax.ShapeDtypeStruct((N_NODES, D), jnp.float32),
    )(oa, ob, oc, W0, W1, W2, bias)


def kernel(x, edge_index, edge_type, W0, b0, W1, b1, W2, b2):
    rows = edge_index[0].astype(jnp.int32)
    cols = edge_index[1].astype(jnp.int32)
    et = edge_type.astype(jnp.int32)
    x = x.astype(jnp.float32)

    h0 = _make_hop(0)(x, rows, cols, et)
    h1, l1, f1, c1 = _make_hop(1, "save")(x, rows, cols, et)
    h2, l2, f2, c2 = _make_hop(2, "save")(x, rows, cols, et)
    o0 = _make_hop(1, "use")(h0, l1, f1, c1)
    o1 = _make_hop(2, "use")(h1, l2, f2, c2)
    o2 = _make_hop(3)(h2, rows, cols, et)
    return _combine(o0, o1, o2, W0, b0, W1, b1, W2, b2)
